# baseline (speedup 1.0000x reference)
import jax
import jax.numpy as jnp
from jax import lax
from jax.experimental import pallas as pl
from jax.experimental.pallas import tpu as pltpu
from jax.experimental.pallas import tpu_sc as plsc

VOCAB = 100000
EMBED = 64
MAX_LEN = 200
BATCH = 4096

G = 100
NGROUPS = BATCH * MAX_LEN // G
NW = 32
GROUPS_PER_W = NGROUPS // NW
K = 4
CHUNKS = GROUPS_PER_W // K
NBUF = 4


def _body(in_ref, tok_ref, pos_ref, out_ref, idx_v, rows_v, *sems):
    semA = sems[0:NBUF]
    semB = sems[NBUF:2 * NBUF]
    semC = sems[2 * NBUF:3 * NBUF]
    cid = lax.axis_index("c")
    sid = lax.axis_index("s")
    wid = sid * 2 + cid
    base_group = wid * GROUPS_PER_W

    def issue_a(c, b):
        row = base_group + c * K
        pltpu.async_copy(in_ref.at[pl.ds(row, K)], idx_v.at[b], semA[b])
        for j in range(K):
            pltpu.async_copy(pos_ref.at[wid, j], rows_v.at[b, j], semA[b])

    def wait_a(b):
        pltpu.make_async_copy(in_ref.at[pl.ds(0, K)], idx_v.at[b],
                              semA[b]).wait()
        for j in range(K):
            pltpu.make_async_copy(pos_ref.at[0, 0], rows_v.at[b, j],
                                  semA[b]).wait()

    def issue_b(b):
        for j in range(K):
            pltpu.async_copy(tok_ref.at[idx_v.at[b, j]], rows_v.at[b, j],
                             semB[b], add=True)

    def wait_b(b):
        for j in range(K):
            pltpu.make_async_copy(tok_ref.at[idx_v.at[b, j]], rows_v.at[b, j],
                                  semB[b]).wait()

    def issue_c(c, b):
        row = base_group + c * K
        pltpu.async_copy(rows_v.at[b], out_ref.at[pl.ds(row, K)], semC[b])

    def wait_c(b):
        pltpu.make_async_copy(rows_v.at[b], out_ref.at[pl.ds(0, K)],
                              semC[b]).wait()

    issue_a(0, 0)
    issue_a(1, 1)

    @pl.loop(0, CHUNKS, step=NBUF)
    def _outer(g):
        for b in range(NBUF):
            c = g + b
            wait_a(b)
            issue_b(b)
            bp = (b - 1) % NBUF
            if b == 0:
                @pl.when(g > 0)
                def _store_prev():
                    wait_b(bp)
                    issue_c(c - 1, bp)
            else:
                wait_b(bp)
                issue_c(c - 1, bp)
            bn = (b + 2) % NBUF

            def _prefill_next():
                if b >= 2:
                    wait_c(bn)
                else:
                    @pl.when(g > 0)
                    def _drain_store():
                        wait_c(bn)
                issue_a(c + 2, bn)

            if b >= 2:
                @pl.when(c + 2 < CHUNKS)
                def _guarded_prefill():
                    _prefill_next()
            else:
                _prefill_next()

    last = CHUNKS - 1
    wait_b(last % NBUF)
    issue_c(last, last % NBUF)
    for b in range(NBUF):
        wait_c(b)


def kernel(input_seq, token_table, pos_table):
    idx2d = input_seq.astype(jnp.int32).reshape(NGROUPS, G)
    pat = jnp.tile(pos_table.reshape(2, G, EMBED), (K // 2, 1, 1))
    pos_rep = jnp.tile(pat[None], (NW, 1, 1, 1))

    mesh = plsc.VectorSubcoreMesh(core_axis_name="c", subcore_axis_name="s")
    run = pl.kernel(
        _body,
        out_type=jax.ShapeDtypeStruct((NGROUPS, G, EMBED), jnp.float32),
        mesh=mesh,
        scratch_types=[
            pltpu.VMEM((NBUF, K, G), jnp.int32),
            pltpu.VMEM((NBUF, K, G, EMBED), jnp.float32),
        ] + [pltpu.SemaphoreType.DMA] * (3 * NBUF),
        compiler_params=pltpu.CompilerParams(use_tc_tiling_on_sc=False),
    )
    out = run(idx2d, token_table, pos_rep)
    return out.reshape(BATCH, MAX_LEN, EMBED)


# speedup vs baseline: 1.1089x; 1.1089x over previous
import jax
import jax.numpy as jnp
from jax import lax
from jax.experimental import pallas as pl
from jax.experimental.pallas import tpu as pltpu
from jax.experimental.pallas import tpu_sc as plsc

VOCAB = 100000
EMBED = 64
MAX_LEN = 200
BATCH = 4096

G = 100
NGROUPS = BATCH * MAX_LEN // G
NW = 32
GROUPS_PER_W = NGROUPS // NW
K = 4
CHUNKS = GROUPS_PER_W // K
NBUF = 4


def _body(in_ref, tok_ref, pos_ref, out_ref, idx_v, rows_v, pat_v, *sems):
    semA = sems[0:NBUF]
    semB = sems[NBUF:2 * NBUF]
    semC = sems[2 * NBUF:3 * NBUF]
    cid = lax.axis_index("c")
    sid = lax.axis_index("s")
    wid = sid * 2 + cid
    base_group = wid * GROUPS_PER_W

    pltpu.sync_copy(pos_ref.at[wid], pat_v)

    def issue_a(c, b):
        row = base_group + c * K
        pltpu.async_copy(in_ref.at[pl.ds(row, K)], idx_v.at[b], semA[b])

    def wait_a(b):
        pltpu.make_async_copy(in_ref.at[pl.ds(0, K)], idx_v.at[b],
                              semA[b]).wait()

    def issue_b(b):
        for j in range(K):
            pltpu.async_copy(tok_ref.at[idx_v.at[b, j]], rows_v.at[b, j],
                             semB[b], add=False)

    def add_pos(b):
        @pl.loop(0, G)
        def _g(gi):
            for k in range(K):
                for e in range(EMBED // 16):
                    sl = pl.ds(e * 16, 16)
                    plsc.addupdate(rows_v.at[b, k, gi, sl], pat_v[k, gi, sl])

    def wait_b(b):
        for j in range(K):
            pltpu.make_async_copy(tok_ref.at[idx_v.at[b, j]], rows_v.at[b, j],
                                  semB[b]).wait()

    def issue_c(c, b):
        row = base_group + c * K
        pltpu.async_copy(rows_v.at[b], out_ref.at[pl.ds(row, K)], semC[b])

    def wait_c(b):
        pltpu.make_async_copy(rows_v.at[b], out_ref.at[pl.ds(0, K)],
                              semC[b]).wait()

    issue_a(0, 0)
    issue_a(1, 1)

    @pl.loop(0, CHUNKS, step=NBUF)
    def _outer(g):
        for b in range(NBUF):
            c = g + b
            wait_a(b)
            issue_b(b)
            bp = (b - 1) % NBUF
            if b == 0:
                @pl.when(g > 0)
                def _store_prev():
                    wait_b(bp)
                    add_pos(bp)
                    issue_c(c - 1, bp)
            else:
                wait_b(bp)
                add_pos(bp)
                issue_c(c - 1, bp)
            bn = (b + 2) % NBUF

            def _prefill_next():
                if b >= 2:
                    wait_c(bn)
                else:
                    @pl.when(g > 0)
                    def _drain_store():
                        wait_c(bn)
                issue_a(c + 2, bn)

            if b >= 2:
                @pl.when(c + 2 < CHUNKS)
                def _guarded_prefill():
                    _prefill_next()
            else:
                _prefill_next()

    last = CHUNKS - 1
    wait_b(last % NBUF)
    add_pos(last % NBUF)
    issue_c(last, last % NBUF)
    for b in range(NBUF):
        wait_c(b)


def kernel(input_seq, token_table, pos_table):
    idx2d = input_seq.astype(jnp.int32).reshape(NGROUPS, G)
    pat = jnp.tile(pos_table.reshape(2, G, EMBED), (K // 2, 1, 1))
    pos_rep = jnp.tile(pat[None], (NW, 1, 1, 1))

    mesh = plsc.VectorSubcoreMesh(core_axis_name="c", subcore_axis_name="s")
    run = pl.kernel(
        _body,
        out_type=jax.ShapeDtypeStruct((NGROUPS, G, EMBED), jnp.float32),
        mesh=mesh,
        scratch_types=[
            pltpu.VMEM((NBUF, K, G), jnp.int32),
            pltpu.VMEM((NBUF, K, G, EMBED), jnp.float32),
            pltpu.VMEM((K, G, EMBED), jnp.float32),
        ] + [pltpu.SemaphoreType.DMA] * (3 * NBUF),
        compiler_params=pltpu.CompilerParams(use_tc_tiling_on_sc=False),
    )
    out = run(idx2d, token_table, pos_rep)
    return out.reshape(BATCH, MAX_LEN, EMBED)
